# baseline (speedup 1.0000x reference)
import functools

import jax
import jax.numpy as jnp
from jax import lax
from jax.experimental import pallas as pl
from jax.experimental.pallas import tpu as pltpu
from jax.experimental.pallas import tpu_sc as plsc

EMB_DIM = 64
NUM_CORES = 2
NUM_SUBCORES = 16
NUM_WORKERS = NUM_CORES * NUM_SUBCORES
BTILE = 128
LANES = 16


@functools.partial(jax.jit, static_argnames=("batch", "seq"))
def _emb_gather(ids_flat, table, batch, seq):
    n_btiles = batch // BTILE
    rows_per_worker = BTILE * seq
    mesh = plsc.VectorSubcoreMesh(
        core_axis_name="c", subcore_axis_name="s",
        num_cores=NUM_CORES, num_subcores=NUM_SUBCORES)

    @functools.partial(
        pl.kernel,
        out_type=jax.ShapeDtypeStruct(
            (seq, EMB_DIM // 8, n_btiles, 8, BTILE), jnp.float32),
        mesh=mesh,
        scratch_types=[
            pltpu.VMEM((rows_per_worker,), jnp.int32),
            pltpu.VMEM((rows_per_worker,), jnp.int32),
            [pltpu.VMEM((BTILE, EMB_DIM), jnp.float32) for _ in range(2)],
            [pltpu.VMEM((EMB_DIM // 8, 8, BTILE), jnp.float32) for _ in range(2)],
            [pltpu.SemaphoreType.DMA for _ in range(2)],
            [pltpu.SemaphoreType.DMA for _ in range(2)],
        ],
        compiler_params=pltpu.CompilerParams(
            use_tc_tiling_on_sc=False, needs_layout_passes=False),
    )
    def gather_kernel(ids_hbm, table_hbm, out_hbm,
                      slab_v, idxc_v, rows, outb, gsem, wsem):
        wid = lax.axis_index("s") * NUM_CORES + lax.axis_index("c")
        base = wid * rows_per_worker
        lane = jax.lax.iota(jnp.int32, LANES)

        pltpu.sync_copy(ids_hbm.at[pl.ds(base, rows_per_worker)], slab_v)

        def comp_body(s, carry):
            for k in range(BTILE // LANES):
                src = (16 * k + lane) * seq + s
                v = plsc.load_gather(slab_v, [src])
                idxc_v[pl.ds(s * BTILE + 16 * k, LANES)] = v
            return carry

        lax.fori_loop(0, seq, comp_body, 0)

        def gcp(s, b):
            return pltpu.make_async_copy(
                table_hbm.at[idxc_v.at[pl.ds(s * BTILE, BTILE)]],
                rows[b], gsem[b])

        def wcp(s, b):
            return pltpu.make_async_copy(
                outb[b], out_hbm.at[s, :, wid], wsem[b])

        def transpose(b):
            rv, ov = rows[b], outb[b]

            def tbody(e, carry):
                e1 = e // 8
                e2 = e - 8 * e1
                ecol = lane * 0 + e
                for k in range(BTILE // LANES):
                    v = plsc.load_gather(rv, [16 * k + lane, ecol])
                    ov[e1, e2, pl.ds(16 * k, LANES)] = v
                return carry

            lax.fori_loop(0, EMB_DIM, tbody, 0)

        for b in range(2):
            gcp(b, b).start()

        for b in range(2):
            gcp(b, b).wait()
            transpose(b)
            wcp(b, b).start()
            gcp(b + 2, b).start()

        def sbody(p, carry):
            s0 = p * 2
            for b in range(2):
                s = s0 + b
                gcp(s, b).wait()
                wcp(s - 2, b).wait()
                transpose(b)
                wcp(s, b).start()
                gcp(s + 2, b).start()
            return carry

        lax.fori_loop(1, seq // 2 - 1, sbody, 0)

        for b in range(2):
            s = seq - 2 + b
            gcp(s, b).wait()
            wcp(s - 2, b).wait()
            transpose(b)
            wcp(s, b).start()
        for b in range(2):
            wcp(seq - 2 + b, b).wait()

    return gather_kernel(ids_flat, table)


def kernel(input_ids, emb_matrix):
    batch, seq = input_ids.shape
    ids_flat = input_ids.reshape(-1).astype(jnp.int32)
    out5 = _emb_gather(ids_flat, emb_matrix, batch, seq)
    return out5.transpose(2, 4, 0, 1, 3).reshape(batch, seq, EMB_DIM)


# speedup vs baseline: 1.8038x; 1.8038x over previous
import functools

import jax
import jax.numpy as jnp
from jax import lax
from jax.experimental import pallas as pl
from jax.experimental.pallas import tpu as pltpu
from jax.experimental.pallas import tpu_sc as plsc

EMB_DIM = 64
NUM_CORES = 2
NUM_SUBCORES = 16
NUM_WORKERS = NUM_CORES * NUM_SUBCORES
BTILE = 128
LANES = 16


@functools.partial(jax.jit, static_argnames=("batch", "seq"))
def _emb_gather(ids_flat, table, batch, seq):
    n_btiles = batch // BTILE
    rows_per_worker = BTILE * seq
    mesh = plsc.VectorSubcoreMesh(
        core_axis_name="c", subcore_axis_name="s",
        num_cores=NUM_CORES, num_subcores=NUM_SUBCORES)

    @functools.partial(
        pl.kernel,
        out_type=jax.ShapeDtypeStruct(
            (seq, EMB_DIM // 8, n_btiles, 8, BTILE), jnp.float32),
        mesh=mesh,
        scratch_types=[
            pltpu.VMEM((rows_per_worker,), jnp.int32),
            pltpu.VMEM((rows_per_worker,), jnp.int32),
            [pltpu.VMEM((BTILE, EMB_DIM), jnp.float32) for _ in range(2)],
            [pltpu.VMEM((EMB_DIM // 8, 8, BTILE), jnp.float32) for _ in range(2)],
            [pltpu.SemaphoreType.DMA for _ in range(2)],
            [pltpu.SemaphoreType.DMA for _ in range(2)],
        ],
        compiler_params=pltpu.CompilerParams(
            use_tc_tiling_on_sc=False, needs_layout_passes=False),
    )
    def gather_kernel(ids_hbm, table_hbm, out_hbm,
                      slab_v, idxc_v, rows, outb, gsem, wsem):
        wid = lax.axis_index("s") * NUM_CORES + lax.axis_index("c")
        base = wid * rows_per_worker
        lane = jax.lax.iota(jnp.int32, LANES)
        klane = [lane + 16 * k for k in range(BTILE // LANES)]
        kseq = [(lane + 16 * k) * seq for k in range(BTILE // LANES)]

        pltpu.sync_copy(ids_hbm.at[pl.ds(base, rows_per_worker)], slab_v)

        @plsc.parallel_loop(0, seq, unroll=2)
        def comp_body(s):
            svec = jnp.full((LANES,), s, jnp.int32)
            sb = s * BTILE
            for k in range(BTILE // LANES):
                v = plsc.load_gather(slab_v, [kseq[k] + svec])
                idxc_v[pl.ds(sb + 16 * k, LANES)] = v

        def gcp(s, b):
            return pltpu.make_async_copy(
                table_hbm.at[idxc_v.at[pl.ds(s * BTILE, BTILE)]],
                rows[b], gsem[b])

        def wcp(s, b):
            return pltpu.make_async_copy(
                outb[b], out_hbm.at[s, :, wid], wsem[b])

        def transpose(b):
            rv, ov = rows[b], outb[b]

            @plsc.parallel_loop(0, EMB_DIM, unroll=2)
            def tbody(e):
                e1 = lax.shift_right_logical(e, 3)
                e2 = lax.bitwise_and(e, 7)
                ecol = jnp.full((LANES,), e, jnp.int32)
                for k in range(BTILE // LANES):
                    v = plsc.load_gather(rv, [klane[k], ecol])
                    ov[e1, e2, pl.ds(16 * k, LANES)] = v

        for b in range(2):
            gcp(b, b).start()

        for b in range(2):
            gcp(b, b).wait()
            transpose(b)
            wcp(b, b).start()
            gcp(b + 2, b).start()

        def sbody(p, carry):
            s0 = p * 2
            for b in range(2):
                s = s0 + b
                gcp(s, b).wait()
                wcp(s - 2, b).wait()
                transpose(b)
                wcp(s, b).start()
                gcp(s + 2, b).start()
            return carry

        lax.fori_loop(1, seq // 2 - 1, sbody, 0)

        for b in range(2):
            s = seq - 2 + b
            gcp(s, b).wait()
            wcp(s - 2, b).wait()
            transpose(b)
            wcp(s, b).start()
        for b in range(2):
            wcp(seq - 2 + b, b).wait()

    return gather_kernel(ids_flat, table)


def kernel(input_ids, emb_matrix):
    batch, seq = input_ids.shape
    ids_flat = input_ids.reshape(-1).astype(jnp.int32)
    out5 = _emb_gather(ids_flat, emb_matrix, batch, seq)
    return out5.transpose(2, 4, 0, 1, 3).reshape(batch, seq, EMB_DIM)


# speedup vs baseline: 4.9716x; 2.7561x over previous
import functools

import jax
import jax.numpy as jnp
from jax import lax
from jax.experimental import pallas as pl
from jax.experimental.pallas import tpu as pltpu
from jax.experimental.pallas import tpu_sc as plsc

EMB_DIM = 64
NUM_CORES = 2
NUM_SUBCORES = 16
NUM_WORKERS = NUM_CORES * NUM_SUBCORES
BTILE = 128
LANES = 16


@functools.partial(jax.jit, static_argnames=("batch", "seq"))
def _emb_gather(ids_flat, table, batch, seq):
    n_btiles = batch // BTILE
    rows_per_worker = BTILE * seq
    mesh = plsc.VectorSubcoreMesh(
        core_axis_name="c", subcore_axis_name="s",
        num_cores=NUM_CORES, num_subcores=NUM_SUBCORES)

    @functools.partial(
        pl.kernel,
        out_type=jax.ShapeDtypeStruct(
            (seq, EMB_DIM // 8, n_btiles, 8, BTILE), jnp.float32),
        mesh=mesh,
        scratch_types=[
            pltpu.VMEM((rows_per_worker,), jnp.int32),
            pltpu.VMEM((rows_per_worker,), jnp.int32),
            [pltpu.VMEM((BTILE, EMB_DIM), jnp.float32) for _ in range(2)],
            [pltpu.VMEM((EMB_DIM // 8, 8, BTILE + 1), jnp.float32) for _ in range(2)],
            [pltpu.SemaphoreType.DMA for _ in range(2)],
            [pltpu.SemaphoreType.DMA for _ in range(2)],
        ],
        compiler_params=pltpu.CompilerParams(
            use_tc_tiling_on_sc=False, needs_layout_passes=False),
    )
    def gather_kernel(ids_hbm, table_hbm, out_hbm,
                      slab_v, idxc_v, rows, outb, gsem, wsem):
        wid = lax.axis_index("s") * NUM_CORES + lax.axis_index("c")
        base = wid * rows_per_worker
        lane = jax.lax.iota(jnp.int32, LANES)
        klane = [lane + 16 * k for k in range(BTILE // LANES)]
        kseq = [(lane + 16 * k) * seq for k in range(BTILE // LANES)]

        pltpu.sync_copy(ids_hbm.at[pl.ds(base, rows_per_worker)], slab_v)

        @plsc.parallel_loop(0, seq, unroll=2)
        def comp_body(s):
            svec = jnp.full((LANES,), s, jnp.int32)
            sb = s * BTILE
            for k in range(BTILE // LANES):
                v = plsc.load_gather(slab_v, [kseq[k] + svec])
                idxc_v[pl.ds(sb + 16 * k, LANES)] = v

        def gcp(s, b):
            return pltpu.make_async_copy(
                table_hbm.at[idxc_v.at[pl.ds(s * BTILE, BTILE)]],
                rows[b], gsem[b])

        def wcp(s, b):
            return pltpu.make_async_copy(
                outb[b].at[:, :, pl.ds(0, BTILE)], out_hbm.at[s, :, wid],
                wsem[b])

        e1c = [lax.shift_right_logical(klane[j], 3)
               for j in range(EMB_DIM // LANES)]
        e2c = [lax.bitwise_and(klane[j], 7) for j in range(EMB_DIM // LANES)]

        def transpose(b):
            rv, ov = rows[b], outb[b]

            @plsc.parallel_loop(0, BTILE, unroll=2)
            def tbody(bb):
                bvec = jnp.full((LANES,), bb, jnp.int32)
                for j in range(EMB_DIM // LANES):
                    v = rv[bb, pl.ds(16 * j, LANES)]
                    plsc.store_scatter(ov, [e1c[j], e2c[j], bvec], v)

        for b in range(2):
            gcp(b, b).start()

        for b in range(2):
            gcp(b, b).wait()
            transpose(b)
            wcp(b, b).start()
            gcp(b + 2, b).start()

        def sbody(p, carry):
            s0 = p * 2
            for b in range(2):
                s = s0 + b
                gcp(s, b).wait()
                wcp(s - 2, b).wait()
                transpose(b)
                wcp(s, b).start()
                gcp(s + 2, b).start()
            return carry

        lax.fori_loop(1, seq // 2 - 1, sbody, 0)

        for b in range(2):
            s = seq - 2 + b
            gcp(s, b).wait()
            wcp(s - 2, b).wait()
            transpose(b)
            wcp(s, b).start()
        for b in range(2):
            wcp(seq - 2 + b, b).wait()

    return gather_kernel(ids_flat, table)


def kernel(input_ids, emb_matrix):
    batch, seq = input_ids.shape
    ids_flat = input_ids.reshape(-1).astype(jnp.int32)
    out5 = _emb_gather(ids_flat, emb_matrix, batch, seq)
    return out5.transpose(2, 4, 0, 1, 3).reshape(batch, seq, EMB_DIM)


# speedup vs baseline: 5.8777x; 1.1823x over previous
import functools

import jax
import jax.numpy as jnp
from jax import lax
from jax.experimental import pallas as pl
from jax.experimental.pallas import tpu as pltpu
from jax.experimental.pallas import tpu_sc as plsc

EMB_DIM = 64
NUM_CORES = 2
NUM_SUBCORES = 16
NUM_WORKERS = NUM_CORES * NUM_SUBCORES
BTILE = 128
LANES = 16
NBUF = 4


@functools.partial(jax.jit, static_argnames=("batch", "seq"))
def _emb_gather(ids_t, table, batch, seq):
    n_btiles = batch // BTILE
    mesh = plsc.VectorSubcoreMesh(
        core_axis_name="c", subcore_axis_name="s",
        num_cores=NUM_CORES, num_subcores=NUM_SUBCORES)

    @functools.partial(
        pl.kernel,
        out_type=jax.ShapeDtypeStruct(
            (seq, EMB_DIM // 8, n_btiles, 8, BTILE), jnp.float32),
        mesh=mesh,
        scratch_types=[
            pltpu.VMEM((seq, BTILE), jnp.int32),
            [pltpu.VMEM((BTILE, EMB_DIM), jnp.float32) for _ in range(NBUF)],
            [pltpu.VMEM((EMB_DIM // 8, 8, BTILE + 1), jnp.float32)
             for _ in range(NBUF)],
            [pltpu.SemaphoreType.DMA for _ in range(NBUF)],
            [pltpu.SemaphoreType.DMA for _ in range(NBUF)],
        ],
        compiler_params=pltpu.CompilerParams(
            use_tc_tiling_on_sc=False, needs_layout_passes=False),
    )
    def gather_kernel(ids_hbm, table_hbm, out_hbm,
                      slab_v, rows, outb, gsem, wsem):
        wid = lax.axis_index("s") * NUM_CORES + lax.axis_index("c")
        lane = jax.lax.iota(jnp.int32, LANES)
        klane = [lane + 16 * k for k in range(BTILE // LANES)]
        e1c = [lax.shift_right_logical(klane[j], 3)
               for j in range(EMB_DIM // LANES)]
        e2c = [lax.bitwise_and(klane[j], 7) for j in range(EMB_DIM // LANES)]

        pltpu.sync_copy(ids_hbm.at[:, pl.ds(wid * BTILE, BTILE)], slab_v)

        def gcp(s, b):
            return pltpu.make_async_copy(
                table_hbm.at[slab_v.at[s]], rows[b], gsem[b])

        def wcp(s, b):
            return pltpu.make_async_copy(
                outb[b].at[:, :, pl.ds(0, BTILE)], out_hbm.at[s, :, wid],
                wsem[b])

        def transpose(b):
            rv, ov = rows[b], outb[b]

            @plsc.parallel_loop(0, BTILE, unroll=4)
            def tbody(bb):
                bvec = jnp.full((LANES,), bb, jnp.int32)
                for j in range(EMB_DIM // LANES):
                    v = rv[bb, pl.ds(16 * j, LANES)]
                    plsc.store_scatter(ov, [e1c[j], e2c[j], bvec], v)

        for b in range(NBUF):
            gcp(b, b).start()

        for b in range(NBUF):
            gcp(b, b).wait()
            transpose(b)
            wcp(b, b).start()
            gcp(b + NBUF, b).start()

        def sbody(q, carry):
            s0 = q * NBUF
            for b in range(NBUF):
                s = s0 + b
                gcp(s, b).wait()
                wcp(s - NBUF, b).wait()
                transpose(b)
                wcp(s, b).start()
                gcp(s + NBUF, b).start()
            return carry

        lax.fori_loop(1, seq // NBUF - 1, sbody, 0)

        for b in range(NBUF):
            s = seq - NBUF + b
            gcp(s, b).wait()
            wcp(s - NBUF, b).wait()
            transpose(b)
            wcp(s, b).start()
        for b in range(NBUF):
            wcp(seq - NBUF + b, b).wait()

    return gather_kernel(ids_t, table)


def kernel(input_ids, emb_matrix):
    batch, seq = input_ids.shape
    ids_t = input_ids.T.astype(jnp.int32)
    out5 = _emb_gather(ids_t, emb_matrix, batch, seq)
    return out5.transpose(2, 4, 0, 1, 3).reshape(batch, seq, EMB_DIM)
